# baseline (speedup 1.0000x reference)
import jax
import jax.numpy as jnp
from jax.experimental import pallas as pl

_BATCH = 4
_SEQ = 4096
_DIM = 4096
_NE = 64
_EPS = 1e-6
_SBLK = 256
_CAP = float(int(1.0 * _SEQ / _NE))


def _gate_body(x_ref, w_ref, b_ref, out_ref):
    x2 = x_ref[...].reshape(_BATCH * _SBLK, _DIM)
    logits = jax.lax.dot_general(
        x2, w_ref[...],
        (((1,), (1,)), ((), ())),
        preferred_element_type=jnp.float32,
    ) + b_ref[...]

    m = jnp.max(logits, axis=-1, keepdims=True)
    e = jnp.exp(logits - m)
    sm = e / jnp.sum(e, axis=-1, keepdims=True)

    iota = jax.lax.broadcasted_iota(jnp.int32, logits.shape, 1)
    is1 = logits == m
    idx1 = jnp.min(jnp.where(is1, iota, _NE), axis=-1, keepdims=True)
    mask1 = iota == idx1
    l2 = jnp.where(mask1, -jnp.inf, logits)
    m2 = jnp.max(l2, axis=-1, keepdims=True)
    is2 = l2 == m2
    idx2 = jnp.min(jnp.where(is2, iota, _NE), axis=-1, keepdims=True)
    mask = mask1 | (iota == idx2)

    masked = jnp.where(mask, sm, 0.0).reshape(_BATCH, _SBLK, _NE)
    denom = jnp.sum(masked, axis=0, keepdims=True) + _EPS
    out_ref[...] = masked / denom * _CAP


def kernel(x, W, b):
    b2 = b.reshape(1, _NE)
    out = pl.pallas_call(
        _gate_body,
        grid=(_SEQ // _SBLK,),
        in_specs=[
            pl.BlockSpec((_BATCH, _SBLK, _DIM), lambda i: (0, i, 0)),
            pl.BlockSpec((_NE, _DIM), lambda i: (0, 0)),
            pl.BlockSpec((1, _NE), lambda i: (0, 0)),
        ],
        out_specs=pl.BlockSpec((_BATCH, _SBLK, _NE), lambda i: (0, i, 0)),
        out_shape=jax.ShapeDtypeStruct((_BATCH, _SEQ, _NE), jnp.float32),
    )(x, W, b2)
    return out


# speedup vs baseline: 1.0895x; 1.0895x over previous
import jax
import jax.numpy as jnp
from jax.experimental import pallas as pl

_BATCH = 4
_SEQ = 4096
_DIM = 4096
_NE = 64
_EPS = 1e-6
_SBLK = 256
_CAP = float(int(1.0 * _SEQ / _NE))


def _gate_body(x_ref, w_ref, b_ref, out_ref):
    x2 = x_ref[...].reshape(_BATCH * _SBLK, _DIM)
    logits = jax.lax.dot_general(
        x2, w_ref[...],
        (((1,), (1,)), ((), ())),
        preferred_element_type=jnp.float32,
    ) + b_ref[...]

    m = jnp.max(logits, axis=-1, keepdims=True)
    e = jnp.exp(logits - m)
    sm = e / jnp.sum(e, axis=-1, keepdims=True)

    iota = jax.lax.broadcasted_iota(jnp.int32, logits.shape, 1)
    is1 = logits == m
    idx1 = jnp.min(jnp.where(is1, iota, _NE), axis=-1, keepdims=True)
    mask1 = iota == idx1
    l2 = jnp.where(mask1, -jnp.inf, logits)
    m2 = jnp.max(l2, axis=-1, keepdims=True)
    is2 = l2 == m2
    idx2 = jnp.min(jnp.where(is2, iota, _NE), axis=-1, keepdims=True)
    mask = mask1 | (iota == idx2)

    masked = jnp.where(mask, sm, 0.0).reshape(_BATCH, _SBLK, _NE)
    denom = jnp.sum(masked, axis=0, keepdims=True) + _EPS
    res = masked / denom * _CAP
    out_ref[...] = jnp.transpose(res, (0, 2, 1))


def kernel(x, W, b):
    b2 = b.reshape(1, _NE)
    out_t = pl.pallas_call(
        _gate_body,
        grid=(_SEQ // _SBLK,),
        in_specs=[
            pl.BlockSpec((_BATCH, _SBLK, _DIM), lambda i: (0, i, 0)),
            pl.BlockSpec((_NE, _DIM), lambda i: (0, 0)),
            pl.BlockSpec((1, _NE), lambda i: (0, 0)),
        ],
        out_specs=pl.BlockSpec((_BATCH, _NE, _SBLK), lambda i: (0, 0, i)),
        out_shape=jax.ShapeDtypeStruct((_BATCH, _NE, _SEQ), jnp.float32),
    )(x, W, b2)
    return jnp.transpose(out_t, (0, 2, 1))
